# baseline (speedup 1.0000x reference)
import functools

import jax
import jax.numpy as jnp
from jax import lax
from jax.experimental import pallas as pl
from jax.experimental.pallas import tpu as pltpu
from jax.experimental.pallas import tpu_sc as plsc

NC = 2
NS = 16
NW = NC * NS

ROWS = 16384
COLS = 50
V = 1000001
D = 64
VP = 1000064
SR = VP // 2
TCOLS_MAIN = 7812
K1N = -(-TCOLS_MAIN // NW)
BPW = ROWS // NW
NCH = COLS * (BPW // 128)

_mesh = plsc.VectorSubcoreMesh(core_axis_name="c", subcore_axis_name="s")
_params = pltpu.CompilerParams(
    use_tc_tiling_on_sc=True, needs_layout_passes=False
)


K1PAIR = (K1N + 1) // 2


@functools.partial(
    pl.kernel,
    mesh=_mesh,
    out_type=jax.ShapeDtypeStruct((SR, 128), jnp.float32),
    scratch_types=[
        pltpu.VMEM((2, D, 128), jnp.float32),
        pltpu.VMEM((2, D, 128), jnp.float32),
        pltpu.VMEM((32, 128), jnp.float32),
        pltpu.SemaphoreType.DMA,
        pltpu.SemaphoreType.DMA,
        pltpu.SemaphoreType.DMA,
        pltpu.SemaphoreType.DMA,
    ],
    compiler_params=_params,
)
def _k1(tT_hbm, tail_hbm, scr_hbm, src_v, dst_v, bounce_v, r0, r1, w0, w1):
    wid = lax.axis_index("s") * NC + lax.axis_index("c")
    rsem = (r0, r1)
    wsem = (w0, w1)
    lanes = lax.iota(jnp.int32, 16)

    @pl.when(wid == 0)
    def _():
        pltpu.sync_copy(tail_hbm, bounce_v)
        pltpu.sync_copy(bounce_v, scr_hbm.at[pl.ds(64 * TCOLS_MAIN, 32)])

    def read_start(ct, p):
        pltpu.async_copy(tT_hbm.at[:, pl.ds(128 * ct, 128)], src_v.at[p], rsem[p])

    def read_wait(ct, p):
        pltpu.make_async_copy(
            tT_hbm.at[:, pl.ds(128 * ct, 128)], src_v.at[p], rsem[p]
        ).wait()

    def write_start(ct, p):
        pltpu.async_copy(dst_v.at[p], scr_hbm.at[pl.ds(64 * ct, 64)], wsem[p])

    def write_wait(ct, p):
        pltpu.make_async_copy(
            dst_v.at[p], scr_hbm.at[pl.ds(64 * ct, 64)], wsem[p]
        ).wait()

    def transpose(p):
        @plsc.parallel_loop(0, 64, unroll=8)
        def _(m):
            for h in range(2):
                col = jnp.full((16,), 2 * m + h, jnp.int32)
                for g in range(4):
                    vals = plsc.load_gather(src_v.at[p], [16 * g + lanes, col])
                    dst_v[p, m, pl.ds(64 * h + 16 * g, 16)] = vals

    def ct_of(j):
        return NW * j + wid

    for p in range(2):
        @pl.when(ct_of(p) < TCOLS_MAIN)
        def _():
            read_start(ct_of(p), p)

    def body(u, carry):
        for p in range(2):
            j = 2 * u + p
            ct = ct_of(j)

            @pl.when(ct < TCOLS_MAIN)
            def _():
                read_wait(ct, p)

                @pl.when(j >= 2)
                def _():
                    write_wait(ct_of(j - 2), p)

                transpose(p)
                write_start(ct, p)
                ct2 = ct_of(j + 2)

                @pl.when(ct2 < TCOLS_MAIN)
                def _():
                    read_start(ct2, p)

        return carry

    lax.fori_loop(0, K1PAIR, body, 0)
    for p in range(2):
        jlast = 2 * (K1PAIR - 1) + p

        @pl.when(ct_of(jlast) < TCOLS_MAIN)
        def _():
            write_wait(ct_of(jlast), p)


@functools.partial(
    pl.kernel,
    mesh=_mesh,
    out_type=jax.ShapeDtypeStruct((COLS, D, ROWS), jnp.float32),
    scratch_types=[
        pltpu.VMEM((56, BPW), jnp.int32),
        pltpu.VMEM((4, 128), jnp.int32),
        pltpu.VMEM((4, 128, 128), jnp.float32),
        pltpu.VMEM((2, D, 128), jnp.float32),
        pltpu.SemaphoreType.DMA,
        pltpu.SemaphoreType.DMA,
        pltpu.SemaphoreType.DMA,
        pltpu.SemaphoreType.DMA,
        pltpu.SemaphoreType.DMA,
        pltpu.SemaphoreType.DMA,
    ],
    compiler_params=_params,
)
def _k2(ids_hbm, scr_hbm, out_hbm, idsv, midx_v, g_v, tout_v,
        g0, g1, g2, g3, s0, s1):
    wid = lax.axis_index("s") * NC + lax.axis_index("c")
    b0 = wid * BPW
    gsem = (g0, g1, g2, g3)
    ssem = (s0, s1)
    lanes = lax.iota(jnp.int32, 16)

    pltpu.sync_copy(ids_hbm.at[:, pl.ds(b0, BPW)], idsv)

    def fire_gather(t, p):
        c = t // 4
        k = t % 4
        for g in range(8):
            v = idsv[c, pl.ds(128 * k + 16 * g, 16)]
            midx_v[p, pl.ds(16 * g, 16)] = lax.shift_right_logical(v, 1)
        pltpu.async_copy(scr_hbm.at[midx_v.at[p]], g_v.at[p], gsem[p])

    def wait_gather(p):
        pltpu.make_async_copy(scr_hbm.at[midx_v.at[p]], g_v.at[p], gsem[p]).wait()

    def transpose_store(t, q, p):
        c = t // 4
        k = t % 4
        rows = []
        pars = []
        for kk in range(8):
            idxv = idsv[c, pl.ds(128 * k + 16 * kk, 16)]
            pars.append(lax.shift_left(idxv & 1, 6))
            rows.append(16 * kk + lanes)

        @plsc.parallel_loop(0, D, unroll=8)
        def _(d):
            for kk in range(8):
                vals = plsc.load_gather(g_v.at[p], [rows[kk], pars[kk] + d])
                tout_v[q, d, pl.ds(16 * kk, 16)] = vals
        pltpu.async_copy(
            tout_v.at[q], out_hbm.at[c].at[:, pl.ds(b0 + 128 * k, 128)], ssem[q]
        )

    def wait_store(t, q):
        c = t // 4
        k = t % 4
        pltpu.make_async_copy(
            tout_v.at[q], out_hbm.at[c].at[:, pl.ds(b0 + 128 * k, 128)], ssem[q]
        ).wait()

    for p in range(4):
        fire_gather(p, p)

    def body(u, carry):
        for p in range(4):
            t = 4 * u + p
            q = p % 2
            wait_gather(p)

            @pl.when(t >= 2)
            def _():
                wait_store(t - 2, q)

            transpose_store(t, q, p)

            @pl.when(t + 4 < NCH)
            def _():
                fire_gather(t + 4, p)

        return carry

    lax.fori_loop(0, NCH // 4, body, 0)
    wait_store(NCH - 2, 0)
    wait_store(NCH - 1, 1)


def kernel(input_ids, table):
    idsT = input_ids.T.astype(jnp.int32)
    ids56 = jnp.pad(idsT, ((0, 6), (0, 0)))
    tT = table.T
    tail = table[TCOLS_MAIN * 128 : TCOLS_MAIN * 128 + 64].reshape(32, 128)
    scratch = _k1(tT, tail)
    out = _k2(ids56, scratch)
    return out.transpose(2, 0, 1)


# speedup vs baseline: 2.8653x; 2.8653x over previous
import functools

import jax
import jax.numpy as jnp
from jax import lax
from jax.experimental import pallas as pl
from jax.experimental.pallas import tpu as pltpu
from jax.experimental.pallas import tpu_sc as plsc

NC = 2
NS = 16
NW = NC * NS

ROWS = 16384
COLS = 50
V = 1000001
D = 64
VP = 1000064
SR = VP // 2
TCOLS_MAIN = 7812
K1N = -(-TCOLS_MAIN // NW)
BPW = ROWS // NW
NCH = COLS * (BPW // 128)

_mesh = plsc.VectorSubcoreMesh(core_axis_name="c", subcore_axis_name="s")
_params = pltpu.CompilerParams(
    use_tc_tiling_on_sc=True, needs_layout_passes=False
)


K1PAIR = (K1N + 1) // 2


@functools.partial(
    pl.kernel,
    mesh=_mesh,
    out_type=jax.ShapeDtypeStruct((SR, 128), jnp.float32),
    scratch_types=[
        pltpu.VMEM((2, D, 128), jnp.float32),
        pltpu.VMEM((2, D, 128), jnp.float32),
        pltpu.VMEM((32, 128), jnp.float32),
        pltpu.SemaphoreType.DMA,
        pltpu.SemaphoreType.DMA,
        pltpu.SemaphoreType.DMA,
        pltpu.SemaphoreType.DMA,
    ],
    compiler_params=_params,
)
def _k1(tT_hbm, tail_hbm, scr_hbm, src_v, dst_v, bounce_v, r0, r1, w0, w1):
    wid = lax.axis_index("s") * NC + lax.axis_index("c")
    rsem = (r0, r1)
    wsem = (w0, w1)
    lanes = lax.iota(jnp.int32, 16)

    @pl.when(wid == 0)
    def _():
        pltpu.sync_copy(tail_hbm, bounce_v)
        pltpu.sync_copy(bounce_v, scr_hbm.at[pl.ds(64 * TCOLS_MAIN, 32)])

    def read_start(ct, p):
        pltpu.async_copy(tT_hbm.at[:, pl.ds(128 * ct, 128)], src_v.at[p], rsem[p])

    def read_wait(ct, p):
        pltpu.make_async_copy(
            tT_hbm.at[:, pl.ds(128 * ct, 128)], src_v.at[p], rsem[p]
        ).wait()

    def write_start(ct, p):
        pltpu.async_copy(dst_v.at[p], scr_hbm.at[pl.ds(64 * ct, 64)], wsem[p])

    def write_wait(ct, p):
        pltpu.make_async_copy(
            dst_v.at[p], scr_hbm.at[pl.ds(64 * ct, 64)], wsem[p]
        ).wait()

    def transpose(p):
        @plsc.parallel_loop(0, 512, unroll=8)
        def _(i):
            r0 = lax.shift_left(lax.shift_right_logical(i, 7), 4)
            c0 = lax.shift_left(lax.shift_right_logical(i, 4) & 7, 4)
            s = i & 15
            cvec = c0 + ((lanes + s) & 15)
            vals = plsc.load_gather(src_v.at[p], [r0 + lanes, cvec])
            mvec = lax.shift_right_logical(cvec, 1)
            hvec = lax.shift_left(cvec & 1, 6) + r0 + lanes
            plsc.store_scatter(dst_v.at[p], [mvec, hvec], vals)

    def ct_of(j):
        return NW * j + wid

    for p in range(2):
        @pl.when(ct_of(p) < TCOLS_MAIN)
        def _():
            read_start(ct_of(p), p)

    def body(u, carry):
        for p in range(2):
            j = 2 * u + p
            ct = ct_of(j)

            @pl.when(ct < TCOLS_MAIN)
            def _():
                read_wait(ct, p)

                @pl.when(j >= 2)
                def _():
                    write_wait(ct_of(j - 2), p)

                transpose(p)
                write_start(ct, p)
                ct2 = ct_of(j + 2)

                @pl.when(ct2 < TCOLS_MAIN)
                def _():
                    read_start(ct2, p)

        return carry

    lax.fori_loop(0, K1PAIR, body, 0)
    for p in range(2):
        jlast = 2 * (K1PAIR - 1) + p

        @pl.when(ct_of(jlast) < TCOLS_MAIN)
        def _():
            write_wait(ct_of(jlast), p)


@functools.partial(
    pl.kernel,
    mesh=_mesh,
    out_type=jax.ShapeDtypeStruct((COLS, D, ROWS), jnp.float32),
    scratch_types=[
        pltpu.VMEM((56, BPW), jnp.int32),
        pltpu.VMEM((4, 128), jnp.int32),
        pltpu.VMEM((4, 128, 128), jnp.float32),
        pltpu.VMEM((2, D, 128), jnp.float32),
        pltpu.SemaphoreType.DMA,
        pltpu.SemaphoreType.DMA,
        pltpu.SemaphoreType.DMA,
        pltpu.SemaphoreType.DMA,
        pltpu.SemaphoreType.DMA,
        pltpu.SemaphoreType.DMA,
    ],
    compiler_params=_params,
)
def _k2(ids_hbm, scr_hbm, out_hbm, idsv, midx_v, g_v, tout_v,
        g0, g1, g2, g3, s0, s1):
    wid = lax.axis_index("s") * NC + lax.axis_index("c")
    b0 = wid * BPW
    gsem = (g0, g1, g2, g3)
    ssem = (s0, s1)
    lanes = lax.iota(jnp.int32, 16)

    pltpu.sync_copy(ids_hbm.at[:, pl.ds(b0, BPW)], idsv)

    def fire_gather(t, p):
        c = t // 4
        k = t % 4
        for g in range(8):
            v = idsv[c, pl.ds(128 * k + 16 * g, 16)]
            midx_v[p, pl.ds(16 * g, 16)] = lax.shift_right_logical(v, 1)
        pltpu.async_copy(scr_hbm.at[midx_v.at[p]], g_v.at[p], gsem[p])

    def wait_gather(p):
        pltpu.make_async_copy(scr_hbm.at[midx_v.at[p]], g_v.at[p], gsem[p]).wait()

    def transpose_store(t, q, p):
        c = t // 4
        k = t % 4
        for kk in range(8):
            idxv = idsv[c, pl.ds(128 * k + 16 * kk, 16)]
            par = lax.shift_left(idxv & 1, 6)
            rows = 16 * kk + lanes

            @plsc.parallel_loop(0, 64, unroll=8)
            def _(i):
                dbase = lax.shift_left(lax.shift_right_logical(i, 4), 4)
                s = i & 15
                dvec = dbase + ((lanes + s) & 15)
                vals = plsc.load_gather(g_v.at[p], [rows, par + dvec])
                plsc.store_scatter(tout_v.at[q], [dvec, rows], vals)
        pltpu.async_copy(
            tout_v.at[q], out_hbm.at[c].at[:, pl.ds(b0 + 128 * k, 128)], ssem[q]
        )

    def wait_store(t, q):
        c = t // 4
        k = t % 4
        pltpu.make_async_copy(
            tout_v.at[q], out_hbm.at[c].at[:, pl.ds(b0 + 128 * k, 128)], ssem[q]
        ).wait()

    for p in range(4):
        fire_gather(p, p)

    def body(u, carry):
        for p in range(4):
            t = 4 * u + p
            q = p % 2
            wait_gather(p)

            @pl.when(t >= 2)
            def _():
                wait_store(t - 2, q)

            transpose_store(t, q, p)

            @pl.when(t + 4 < NCH)
            def _():
                fire_gather(t + 4, p)

        return carry

    lax.fori_loop(0, NCH // 4, body, 0)
    wait_store(NCH - 2, 0)
    wait_store(NCH - 1, 1)


def kernel(input_ids, table):
    idsT = input_ids.T.astype(jnp.int32)
    ids56 = jnp.pad(idsT, ((0, 6), (0, 0)))
    tT = table.T
    tail = table[TCOLS_MAIN * 128 : TCOLS_MAIN * 128 + 64].reshape(32, 128)
    scratch = _k1(tT, tail)
    out = _k2(ids56, scratch)
    return out.transpose(2, 0, 1)
